# baseline (speedup 1.0000x reference)
import functools

import jax
import jax.numpy as jnp
from jax import lax
from jax.experimental import pallas as pl
from jax.experimental.pallas import tpu as pltpu
import jax.experimental.pallas.tpu_sc as plsc

N = 10000
E = 320000
P = 100
D = 128
N_GNN = 7
N_EVAL = 5

N_PAD = 10240
E_PAD = 327680
CHUNK = 128
NBUF = 4
ABUF = 2
N_CHUNKS = E_PAD // (16 * CHUNK)
ROWS_PER_TILE = N_PAD // 16
LCHUNK = 64
L_CHUNKS = E_PAD // (32 * LCHUNK)
X_ROWS = N_PAD // 32
HI = lax.Precision.HIGHEST

_f32 = jnp.float32


@functools.cache
def _sc_kernels():
    mesh = plsc.VectorSubcoreMesh(core_axis_name="c", subcore_axis_name="s",
                                  num_cores=2, num_subcores=16)

    @functools.partial(
        pl.kernel,
        out_type=[
            jax.ShapeDtypeStruct((N_PAD, 16), _f32),
            jax.ShapeDtypeStruct((N_PAD, 16), _f32),
            jax.ShapeDtypeStruct((N_PAD, 16), _f32),
            jax.ShapeDtypeStruct((N_PAD, 16), _f32),
            jax.ShapeDtypeStruct((N_PAD, D), _f32),
        ],
        mesh=mesh,
        scratch_types=[
            pltpu.VMEM_SHARED((N_PAD, 16), _f32),
            pltpu.VMEM_SHARED((N_PAD, 16), _f32),
            pltpu.VMEM((CHUNK, 16), _f32),
            pltpu.VMEM((CHUNK, 16), _f32),
            pltpu.VMEM((NBUF, CHUNK), jnp.int32),
            pltpu.VMEM((NBUF, CHUNK), jnp.int32),
            pltpu.VMEM((LCHUNK,), jnp.int32),
            pltpu.VMEM((LCHUNK, D), _f32),
            pltpu.SemaphoreType.DMA,
        ] + [pltpu.SemaphoreType.DMA] * (4 * NBUF),
    )
    def _sc_prep(nodes, s1, r1, s2, r2, embed,
                 deg_s1, deg_r1, deg_s2, deg_r2, x0,
                 acc_a, acc_b, ones_v, zeros_v, idxa_m, idxb_m,
                 nidx_v, xrows_v, sem, *sems):
        sia = sems[0:NBUF]
        sib = sems[NBUF:2 * NBUF]
        ssa = sems[2 * NBUF:3 * NBUF]
        ssb = sems[3 * NBUF:4 * NBUF]
        c = lax.axis_index("c")
        tile = lax.axis_index("s")

        def fill(i, _):
            ones_v[i, :] = jnp.ones((16,), _f32)
            zeros_v[i, :] = jnp.zeros((16,), _f32)
            return 0
        lax.fori_loop(0, CHUNK, fill, 0)

        base_r = tile * ROWS_PER_TILE
        for k in range(ROWS_PER_TILE // CHUNK):
            pltpu.sync_copy(zeros_v, acc_a.at[pl.ds(base_r + k * CHUNK, CHUNK)])
            pltpu.sync_copy(zeros_v, acc_b.at[pl.ds(base_r + k * CHUNK, CHUNK)])
        plsc.subcore_barrier()

        def deg_pass(sref, rref):
            ebase0 = tile * (N_CHUNKS * CHUNK)

            def outer(it, _):
                for b in range(NBUF):
                    ebase = ebase0 + (it * NBUF + b) * CHUNK

                    @pl.when(it > 0)
                    def _():
                        pltpu.make_async_copy(
                            ones_v, acc_a.at[idxa_m.at[b]], ssa[b]).wait()
                        pltpu.make_async_copy(
                            ones_v, acc_b.at[idxb_m.at[b]], ssb[b]).wait()
                    pltpu.async_copy(sref.at[pl.ds(ebase, CHUNK)],
                                     idxa_m.at[b], sia[b])
                    pltpu.async_copy(rref.at[pl.ds(ebase, CHUNK)],
                                     idxb_m.at[b], sib[b])
                for b in range(NBUF):
                    ebase = ebase0 + (it * NBUF + b) * CHUNK
                    pltpu.make_async_copy(sref.at[pl.ds(ebase, CHUNK)],
                                          idxa_m.at[b], sia[b]).wait()
                    pltpu.async_copy(ones_v, acc_a.at[idxa_m.at[b]], ssa[b],
                                     add=True)
                    pltpu.make_async_copy(rref.at[pl.ds(ebase, CHUNK)],
                                          idxb_m.at[b], sib[b]).wait()
                    pltpu.async_copy(ones_v, acc_b.at[idxb_m.at[b]], ssb[b],
                                     add=True)
                return 0
            lax.fori_loop(0, N_CHUNKS // NBUF, outer, 0)
            for b in range(NBUF):
                pltpu.make_async_copy(ones_v, acc_a.at[idxa_m.at[b]], ssa[b]).wait()
                pltpu.make_async_copy(ones_v, acc_b.at[idxb_m.at[b]], ssb[b]).wait()

        @pl.when(c == 0)
        def _():
            deg_pass(s1, r1)

        @pl.when(c == 1)
        def _():
            deg_pass(s2, r2)

        plsc.subcore_barrier()

        @pl.when(c == 0)
        def _():
            pltpu.sync_copy(acc_a.at[pl.ds(base_r, ROWS_PER_TILE)],
                            deg_s1.at[pl.ds(base_r, ROWS_PER_TILE)])
            pltpu.sync_copy(acc_b.at[pl.ds(base_r, ROWS_PER_TILE)],
                            deg_r1.at[pl.ds(base_r, ROWS_PER_TILE)])

        @pl.when(c == 1)
        def _():
            pltpu.sync_copy(acc_a.at[pl.ds(base_r, ROWS_PER_TILE)],
                            deg_s2.at[pl.ds(base_r, ROWS_PER_TILE)])
            pltpu.sync_copy(acc_b.at[pl.ds(base_r, ROWS_PER_TILE)],
                            deg_r2.at[pl.ds(base_r, ROWS_PER_TILE)])

        w = tile * 2 + c
        for j in range(X_ROWS // LCHUNK):
            nbase = w * X_ROWS + j * LCHUNK
            pltpu.sync_copy(nodes.at[pl.ds(nbase, LCHUNK)], nidx_v)
            pltpu.async_copy(embed.at[nidx_v], xrows_v, sem).wait()
            pltpu.sync_copy(xrows_v, x0.at[pl.ds(nbase, LCHUNK)])

    @functools.partial(
        pl.kernel,
        out_type=[
            jax.ShapeDtypeStruct((N_PAD, D), _f32),
            jax.ShapeDtypeStruct((N_PAD, D), _f32),
        ],
        mesh=mesh,
        scratch_types=[
            pltpu.VMEM_SHARED((N_PAD, D), _f32),
        ] + [pltpu.VMEM((CHUNK, D), _f32)] * ABUF + [
            pltpu.VMEM((ABUF, CHUNK), jnp.int32),
            pltpu.VMEM((ABUF, CHUNK), jnp.int32),
        ] + [pltpu.SemaphoreType.DMA] * (2 * ABUF),
    )
    def _sc_agg(h1, h2, s1, r1, s2, r2, o1, o2, acc, *rest):
        rows = rest[0:ABUF]
        sidx_m = rest[ABUF]
        ridx_m = rest[ABUF + 1]
        sg = rest[ABUF + 2:2 * ABUF + 2]
        ss = rest[2 * ABUF + 2:3 * ABUF + 2]
        c = lax.axis_index("c")
        tile = lax.axis_index("s")

        def fill(i, _):
            for t in range(D // 16):
                rows[0][i, pl.ds(t * 16, 16)] = jnp.zeros((16,), _f32)
            return 0
        lax.fori_loop(0, CHUNK, fill, 0)

        base_r = tile * ROWS_PER_TILE
        for k in range(ROWS_PER_TILE // CHUNK):
            pltpu.sync_copy(rows[0], acc.at[pl.ds(base_r + k * CHUNK, CHUNK)])
        plsc.subcore_barrier()

        def agg(href, sref, rref):
            ebase0 = tile * (N_CHUNKS * CHUNK)

            def outer(it, _):
                for b in range(ABUF):
                    ebase = ebase0 + (it * ABUF + b) * CHUNK

                    @pl.when(it > 0)
                    def _():
                        pltpu.make_async_copy(
                            rows[b], acc.at[ridx_m.at[b]], ss[b]).wait()
                    pltpu.sync_copy(sref.at[pl.ds(ebase, CHUNK)], sidx_m.at[b])
                    pltpu.sync_copy(rref.at[pl.ds(ebase, CHUNK)], ridx_m.at[b])
                    pltpu.async_copy(href.at[sidx_m.at[b]], rows[b], sg[b])
                for b in range(ABUF):
                    pltpu.make_async_copy(
                        href.at[sidx_m.at[b]], rows[b], sg[b]).wait()
                    pltpu.async_copy(rows[b], acc.at[ridx_m.at[b]], ss[b],
                                     add=True)
                return 0
            lax.fori_loop(0, N_CHUNKS // ABUF, outer, 0)
            for b in range(ABUF):
                pltpu.make_async_copy(rows[b], acc.at[ridx_m.at[b]], ss[b]).wait()

        @pl.when(c == 0)
        def _():
            agg(h1, s1, r1)

        @pl.when(c == 1)
        def _():
            agg(h2, s2, r2)

        plsc.subcore_barrier()

        @pl.when(c == 0)
        def _():
            pltpu.sync_copy(acc.at[pl.ds(base_r, ROWS_PER_TILE)],
                            o1.at[pl.ds(base_r, ROWS_PER_TILE)])

        @pl.when(c == 1)
        def _():
            pltpu.sync_copy(acc.at[pl.ds(base_r, ROWS_PER_TILE)],
                            o2.at[pl.ds(base_r, ROWS_PER_TILE)])

    @functools.partial(
        pl.kernel,
        out_type=jax.ShapeDtypeStruct((E_PAD, 16), _f32),
        mesh=mesh,
        scratch_types=[
            pltpu.VMEM((2, LCHUNK), jnp.int32),
            pltpu.VMEM((2, LCHUNK), jnp.int32),
        ] + [pltpu.VMEM((LCHUNK, D), _f32)] * 4 + [
            pltpu.VMEM((LCHUNK, 16), _f32),
            pltpu.VMEM((LCHUNK, 16), _f32),
        ] + [pltpu.SemaphoreType.DMA] * 10,
    )
    def _sc_logits(ln, s1, r1, out, isx_m, irx_m, rs0, rs1, rr0, rr1,
                   ov0, ov1, *sems):
        rs = (rs0, rs1)
        rr = (rr0, rr1)
        ov = (ov0, ov1)
        ii_s = sems[0:2]
        ii_r = sems[2:4]
        gg_s = sems[4:6]
        gg_r = sems[6:8]
        oo = sems[8:10]
        c = lax.axis_index("c")
        tile = lax.axis_index("s")
        w = tile * 2 + c
        ebase0 = w * (L_CHUNKS * LCHUNK)

        def idx_load(j, b):
            ebase = ebase0 + j * LCHUNK
            pltpu.async_copy(s1.at[pl.ds(ebase, LCHUNK)], isx_m.at[b], ii_s[b])
            pltpu.async_copy(r1.at[pl.ds(ebase, LCHUNK)], irx_m.at[b], ii_r[b])

        def idx_wait(j, b):
            ebase = ebase0 + j * LCHUNK
            pltpu.make_async_copy(s1.at[pl.ds(ebase, LCHUNK)], isx_m.at[b],
                                  ii_s[b]).wait()
            pltpu.make_async_copy(r1.at[pl.ds(ebase, LCHUNK)], irx_m.at[b],
                                  ii_r[b]).wait()

        def gathers(b):
            pltpu.async_copy(ln.at[isx_m.at[b]], rs[b], gg_s[b])
            pltpu.async_copy(ln.at[irx_m.at[b]], rr[b], gg_r[b])

        def gathers_wait(b):
            pltpu.make_async_copy(ln.at[isx_m.at[b]], rs[b], gg_s[b]).wait()
            pltpu.make_async_copy(ln.at[irx_m.at[b]], rr[b], gg_r[b]).wait()

        idx_load(0, 0)
        idx_wait(0, 0)
        gathers(0)
        idx_load(1, 1)

        def outer(it, _):
            for b in range(2):
                j2 = it * 2 + b

                @pl.when(j2 < L_CHUNKS - 1)
                def _():
                    idx_wait(j2 + 1, b ^ 1)
                    gathers(b ^ 1)
                gathers_wait(b)

                @pl.when(j2 < L_CHUNKS - 2)
                def _():
                    idx_load(j2 + 2, b)

                @pl.when(j2 >= 2)
                def _():
                    pltpu.make_async_copy(
                        ov[b], out.at[pl.ds(ebase0 + (j2 - 2) * LCHUNK, LCHUNK)],
                        oo[b]).wait()

                def row(i, _):
                    acc = rs[b][i, pl.ds(0, 16)] * rr[b][i, pl.ds(0, 16)]
                    for t in range(1, D // 16):
                        acc = acc + (rs[b][i, pl.ds(t * 16, 16)]
                                     * rr[b][i, pl.ds(t * 16, 16)])
                    ov[b][i, :] = acc
                    return 0
                lax.fori_loop(0, LCHUNK, row, 0)
                pltpu.async_copy(
                    ov[b], out.at[pl.ds(ebase0 + j2 * LCHUNK, LCHUNK)], oo[b])
            return 0
        lax.fori_loop(0, L_CHUNKS // 2, outer, 0)
        for b in range(2):
            pltpu.make_async_copy(
                ov[b], out.at[pl.ds(ebase0, LCHUNK)], oo[b]).wait()

    return _sc_prep, _sc_agg, _sc_logits


def _tc_layer0_body(x0_ref, ds1_ref, dr1_ref, ds2_ref, dr2_ref,
                    W1_ref, b1_ref, W2_ref, b2_ref,
                    h1_ref, h2_ref, is1_ref, is2_ref, ir1_ref, ir2_ref):
    x0 = x0_ref[...]
    shape = x0.shape
    is1 = jnp.broadcast_to(lax.rsqrt(ds1_ref[:, :1] + 1.0), shape)
    is2 = jnp.broadcast_to(lax.rsqrt(ds2_ref[:, :1] + 1.0), shape)
    ir1 = jnp.broadcast_to(lax.rsqrt(dr1_ref[:, :1] + 1.0), shape)
    ir2 = jnp.broadcast_to(lax.rsqrt(dr2_ref[:, :1] + 1.0), shape)
    h1_ref[...] = (jnp.dot(x0, W1_ref[...], precision=HI) + b1_ref[...]) * is1
    h2_ref[...] = (jnp.dot(x0, W2_ref[...], precision=HI) + b2_ref[...]) * is2
    is1_ref[...] = is1
    is2_ref[...] = is2
    ir1_ref[...] = ir1
    ir2_ref[...] = ir2


_BLK = 512
_GRID = N_PAD // _BLK


def _rows(shape=(_BLK, D)):
    return pl.BlockSpec(shape, lambda i: (i, 0))


def _full(shape):
    return pl.BlockSpec(shape, lambda i: (0, 0))


_tc_layer0 = pl.pallas_call(
    _tc_layer0_body,
    grid=(_GRID,),
    in_specs=[_rows(), _rows((_BLK, 16)), _rows((_BLK, 16)), _rows((_BLK, 16)),
              _rows((_BLK, 16)), _full((D, D)), _full((1, D)), _full((D, D)),
              _full((1, D))],
    out_specs=[_rows(), _rows(), _rows(), _rows(), _rows(), _rows()],
    out_shape=[jax.ShapeDtypeStruct((N_PAD, D), _f32)] * 6,
)


def _tc_layer_body(o1_ref, o2_ref, h1_ref, h2_ref, ir1_ref, ir2_ref,
                   is1_ref, is2_ref, WmA_ref, WmG_ref, bm_ref,
                   W1n_ref, b1n_ref, W2n_ref, b2n_ref,
                   h1n_ref, h2n_ref):
    a = (o1_ref[...] + h1_ref[...]) * ir1_ref[...]
    g = (o2_ref[...] + h2_ref[...]) * ir2_ref[...]
    xn = jnp.maximum(jnp.dot(a, WmA_ref[...], precision=HI)
                     + jnp.dot(g, WmG_ref[...], precision=HI) + bm_ref[...], 0.0)
    h1n_ref[...] = (jnp.dot(xn, W1n_ref[...], precision=HI) + b1n_ref[...]) * is1_ref[...]
    h2n_ref[...] = (jnp.dot(xn, W2n_ref[...], precision=HI) + b2n_ref[...]) * is2_ref[...]


_tc_layer = pl.pallas_call(
    _tc_layer_body,
    grid=(_GRID,),
    in_specs=[_rows()] * 8 + [_full((D, D)), _full((D, D)), _full((1, D)),
                              _full((D, D)), _full((1, D)), _full((D, D)),
                              _full((1, D))],
    out_specs=[_rows(), _rows()],
    out_shape=[jax.ShapeDtypeStruct((N_PAD, D), _f32)] * 2,
)


def _tc_final_body(o1_ref, o2_ref, h1_ref, h2_ref, ir1_ref, ir2_ref,
                   WmA_ref, WmG_ref, bm_ref, Wl_ref, bl_ref,
                   x_ref, ln_ref):
    a = (o1_ref[...] + h1_ref[...]) * ir1_ref[...]
    g = (o2_ref[...] + h2_ref[...]) * ir2_ref[...]
    xn = jnp.maximum(jnp.dot(a, WmA_ref[...], precision=HI)
                     + jnp.dot(g, WmG_ref[...], precision=HI) + bm_ref[...], 0.0)
    x_ref[...] = xn
    ln_ref[...] = jnp.dot(xn, Wl_ref[...], precision=HI) + bl_ref[...]


_tc_final = pl.pallas_call(
    _tc_final_body,
    grid=(_GRID,),
    in_specs=[_rows()] * 6 + [_full((D, D)), _full((D, D)), _full((1, D)),
                              _full((D, D)), _full((1, D))],
    out_specs=[_rows(), _rows()],
    out_shape=[jax.ShapeDtypeStruct((N_PAD, D), _f32)] * 2,
)


def _tc_eval_body(x_ref, We_ref, be_ref, Wo_ref, bo_ref, v_ref):
    x = x_ref[...]
    pid = lax.broadcasted_iota(jnp.int32, (D, N_PAD), 0)
    nid = lax.broadcasted_iota(jnp.int32, (D, N_PAD), 1)
    seg = (nid // (N // P) == pid) & (nid % (N // P) != 0)
    v = jnp.dot(seg.astype(_f32), x, precision=HI) * (1.0 / (N // P - 1))
    We = We_ref[...]
    be = be_ref[...]
    for i in range(N_EVAL):
        v = jnp.maximum(
            jnp.dot(v, We[i * D:(i + 1) * D, :], precision=HI) + be[i:i + 1, :],
            0.0)
    v_ref[...] = jnp.tanh(jnp.dot(v, Wo_ref[...], precision=HI) + bo_ref[...])


_tc_eval = pl.pallas_call(
    _tc_eval_body,
    out_shape=jax.ShapeDtypeStruct((D, D), _f32),
)


def _tc_lsum_body(pv_ref, out_ref):
    out_ref[...] = jnp.sum(pv_ref[...], axis=1, keepdims=True)


_tc_lsum = pl.pallas_call(
    _tc_lsum_body,
    grid=(128,),
    in_specs=[pl.BlockSpec((E_PAD // 128, 16), lambda i: (i, 0))],
    out_specs=pl.BlockSpec((E_PAD // 128, 1), lambda i: (i, 0)),
    out_shape=jax.ShapeDtypeStruct((E_PAD, 1), _f32),
)


def kernel(nodes, senders, receivers, grid_senders, grid_receivers, n_node,
           embed, W_conv1, b_conv1, W_conv2, b_conv2, W_mix, b_mix,
           W_logits, b_logits, W_eval, b_eval, W_out, b_out):
    sc_prep, sc_agg, sc_logits = _sc_kernels()

    pad_e = jnp.full((E_PAD - E,), N, jnp.int32)
    s1 = jnp.concatenate([senders, pad_e])
    r1 = jnp.concatenate([receivers, pad_e])
    s2 = jnp.concatenate([grid_senders, pad_e])
    r2 = jnp.concatenate([grid_receivers, pad_e])
    nodes_p = jnp.concatenate([nodes, jnp.zeros((N_PAD - N,), jnp.int32)])

    deg_s1, deg_r1, deg_s2, deg_r2, x0 = sc_prep(nodes_p, s1, r1, s2, r2, embed)

    h1, h2, is1, is2, ir1, ir2 = _tc_layer0(
        x0, deg_s1, deg_r1, deg_s2, deg_r2,
        W_conv1[0], b_conv1[0].reshape(1, D), W_conv2[0], b_conv2[0].reshape(1, D))

    for i in range(N_GNN - 1):
        o1, o2 = sc_agg(h1, h2, s1, r1, s2, r2)
        h1, h2 = _tc_layer(
            o1, o2, h1, h2, ir1, ir2, is1, is2,
            W_mix[i, :D, :], W_mix[i, D:, :], b_mix[i].reshape(1, D),
            W_conv1[i + 1], b_conv1[i + 1].reshape(1, D),
            W_conv2[i + 1], b_conv2[i + 1].reshape(1, D))

    o1, o2 = sc_agg(h1, h2, s1, r1, s2, r2)
    x, ln = _tc_final(
        o1, o2, h1, h2, ir1, ir2,
        W_mix[6, :D, :], W_mix[6, D:, :], b_mix[6].reshape(1, D),
        W_logits, b_logits.reshape(1, D))

    pv = sc_logits(ln, s1, r1)
    logits = _tc_lsum(pv)[:E, 0]

    v = _tc_eval(x, W_eval.reshape(N_EVAL * D, D), b_eval,
                 jnp.pad(W_out, ((0, 0), (0, D - 1))),
                 jnp.pad(b_out.reshape(1, 1), ((0, 0), (0, D - 1))))
    return logits, v[:P, :1]


# speedup vs baseline: 1.0004x; 1.0004x over previous
import functools

import jax
import jax.numpy as jnp
from jax import lax
from jax.experimental import pallas as pl
from jax.experimental.pallas import tpu as pltpu
import jax.experimental.pallas.tpu_sc as plsc

N = 10000
E = 320000
P = 100
D = 128
N_GNN = 7
N_EVAL = 5

N_PAD = 10240
E_PAD = 327680
CHUNK = 128
N_CHUNKS = E_PAD // (16 * CHUNK)
BLKCH = 32
ROWS_PER_TILE = N_PAD // 16
L_CHUNKS = E_PAD // (32 * CHUNK)
X_ROWS = N_PAD // 32
XCH = 64
HI = lax.Precision.HIGHEST

_f32 = jnp.float32


@functools.cache
def _sc_kernels():
    mesh = plsc.VectorSubcoreMesh(core_axis_name="c", subcore_axis_name="s",
                                  num_cores=2, num_subcores=16)

    @functools.partial(
        pl.kernel,
        out_type=[
            jax.ShapeDtypeStruct((N_PAD, 16), _f32),
            jax.ShapeDtypeStruct((N_PAD, 16), _f32),
            jax.ShapeDtypeStruct((N_PAD, 16), _f32),
            jax.ShapeDtypeStruct((N_PAD, 16), _f32),
            jax.ShapeDtypeStruct((N_PAD, D), _f32),
        ],
        mesh=mesh,
        scratch_types=[
            pltpu.VMEM_SHARED((N_PAD, 16), _f32),
            pltpu.VMEM_SHARED((N_PAD, 16), _f32),
            pltpu.VMEM((CHUNK, 16), _f32),
            pltpu.VMEM((CHUNK, 16), _f32),
            pltpu.VMEM((N_CHUNKS, CHUNK), jnp.int32),
            pltpu.VMEM((N_CHUNKS, CHUNK), jnp.int32),
            pltpu.VMEM((XCH,), jnp.int32),
            pltpu.VMEM((XCH, D), _f32),
            pltpu.SemaphoreType.DMA,
            pltpu.SemaphoreType.DMA,
            pltpu.SemaphoreType.DMA,
        ],
    )
    def _sc_prep(nodes, s1, r1, s2, r2, embed,
                 deg_s1, deg_r1, deg_s2, deg_r2, x0,
                 acc_a, acc_b, ones_v, zeros_v, idxa_m, idxb_m,
                 nidx_v, xrows_v, sem, sa, sb):
        c = lax.axis_index("c")
        tile = lax.axis_index("s")

        def fill(i, _):
            ones_v[i, :] = jnp.ones((16,), _f32)
            zeros_v[i, :] = jnp.zeros((16,), _f32)
            return 0
        lax.fori_loop(0, CHUNK, fill, 0)

        base_r = tile * ROWS_PER_TILE
        for k in range(ROWS_PER_TILE // CHUNK):
            pltpu.sync_copy(zeros_v, acc_a.at[pl.ds(base_r + k * CHUNK, CHUNK)])
            pltpu.sync_copy(zeros_v, acc_b.at[pl.ds(base_r + k * CHUNK, CHUNK)])
        plsc.subcore_barrier()

        def deg_pass(sref, rref):
            row0 = tile * N_CHUNKS
            pltpu.sync_copy(sref.at[pl.ds(row0, N_CHUNKS)], idxa_m)
            pltpu.sync_copy(rref.at[pl.ds(row0, N_CHUNKS)], idxb_m)

            def fire(j, _):
                pltpu.async_copy(ones_v, acc_a.at[idxa_m.at[j]], sa, add=True)
                pltpu.async_copy(ones_v, acc_b.at[idxb_m.at[j]], sb, add=True)
                return 0
            lax.fori_loop(0, N_CHUNKS, fire, 0)

            def drain(j, _):
                pltpu.make_async_copy(ones_v, acc_a.at[idxa_m.at[0]], sa).wait()
                pltpu.make_async_copy(ones_v, acc_b.at[idxb_m.at[0]], sb).wait()
                return 0
            lax.fori_loop(0, N_CHUNKS, drain, 0)

        @pl.when(c == 0)
        def _():
            deg_pass(s1, r1)

        @pl.when(c == 1)
        def _():
            deg_pass(s2, r2)

        plsc.subcore_barrier()

        @pl.when(c == 0)
        def _():
            pltpu.sync_copy(acc_a.at[pl.ds(base_r, ROWS_PER_TILE)],
                            deg_s1.at[pl.ds(base_r, ROWS_PER_TILE)])
            pltpu.sync_copy(acc_b.at[pl.ds(base_r, ROWS_PER_TILE)],
                            deg_r1.at[pl.ds(base_r, ROWS_PER_TILE)])

        @pl.when(c == 1)
        def _():
            pltpu.sync_copy(acc_a.at[pl.ds(base_r, ROWS_PER_TILE)],
                            deg_s2.at[pl.ds(base_r, ROWS_PER_TILE)])
            pltpu.sync_copy(acc_b.at[pl.ds(base_r, ROWS_PER_TILE)],
                            deg_r2.at[pl.ds(base_r, ROWS_PER_TILE)])

        w = tile * 2 + c
        for j in range(X_ROWS // XCH):
            nbase = w * X_ROWS + j * XCH
            pltpu.sync_copy(nodes.at[pl.ds(nbase, XCH)], nidx_v)
            pltpu.async_copy(embed.at[nidx_v], xrows_v, sem).wait()
            pltpu.sync_copy(xrows_v, x0.at[pl.ds(nbase, XCH)])

    @functools.partial(
        pl.kernel,
        out_type=[
            jax.ShapeDtypeStruct((N_PAD, D), _f32),
            jax.ShapeDtypeStruct((N_PAD, D), _f32),
        ],
        mesh=mesh,
        scratch_types=[
            pltpu.VMEM_SHARED((N_PAD, D), _f32),
            pltpu.VMEM((CHUNK, D), _f32),
            pltpu.VMEM((CHUNK, D), _f32),
            pltpu.VMEM((BLKCH, CHUNK), jnp.int32),
            pltpu.VMEM((BLKCH, CHUNK), jnp.int32),
            pltpu.SemaphoreType.DMA,
            pltpu.SemaphoreType.DMA,
        ],
    )
    def _sc_agg(h1, h2, s1, r1, s2, r2, o1, o2,
                acc, rw0, rw1, sidx_m, ridx_m, sg0, sg1):
        rows = (rw0, rw1)
        sg = (sg0, sg1)
        c = lax.axis_index("c")
        tile = lax.axis_index("s")

        def fill(i, _):
            for t in range(D // 16):
                rw0[i, pl.ds(t * 16, 16)] = jnp.zeros((16,), _f32)
            return 0
        lax.fori_loop(0, CHUNK, fill, 0)

        base_r = tile * ROWS_PER_TILE
        for k in range(ROWS_PER_TILE // CHUNK):
            pltpu.sync_copy(rw0, acc.at[pl.ds(base_r + k * CHUNK, CHUNK)])
        plsc.subcore_barrier()

        def agg(href, sref, rref):
            row_base = tile * N_CHUNKS

            def blk(bi, _):
                row0 = row_base + bi * BLKCH
                pltpu.sync_copy(sref.at[pl.ds(row0, BLKCH)], sidx_m)
                pltpu.sync_copy(rref.at[pl.ds(row0, BLKCH)], ridx_m)
                pltpu.async_copy(href.at[sidx_m.at[0]], rows[0], sg[0])

                def inner(it, _):
                    for b in range(2):
                        j = it * 2 + b
                        pltpu.make_async_copy(
                            href.at[sidx_m.at[j]], rows[b], sg[b]).wait()

                        @pl.when(j + 1 < BLKCH)
                        def _():
                            pltpu.async_copy(href.at[sidx_m.at[j + 1]],
                                             rows[b ^ 1], sg[b ^ 1])
                        pltpu.sync_copy(rows[b], acc.at[ridx_m.at[j]],
                                        add=True)
                    return 0
                lax.fori_loop(0, BLKCH // 2, inner, 0)
                return 0
            lax.fori_loop(0, N_CHUNKS // BLKCH, blk, 0)

        @pl.when(c == 0)
        def _():
            agg(h1, s1, r1)

        @pl.when(c == 1)
        def _():
            agg(h2, s2, r2)

        plsc.subcore_barrier()

        @pl.when(c == 0)
        def _():
            pltpu.sync_copy(acc.at[pl.ds(base_r, ROWS_PER_TILE)],
                            o1.at[pl.ds(base_r, ROWS_PER_TILE)])

        @pl.when(c == 1)
        def _():
            pltpu.sync_copy(acc.at[pl.ds(base_r, ROWS_PER_TILE)],
                            o2.at[pl.ds(base_r, ROWS_PER_TILE)])

    @functools.partial(
        pl.kernel,
        out_type=jax.ShapeDtypeStruct((E_PAD, 16), _f32),
        mesh=mesh,
        scratch_types=[
            pltpu.VMEM((L_CHUNKS, CHUNK), jnp.int32),
            pltpu.VMEM((L_CHUNKS, CHUNK), jnp.int32),
            pltpu.VMEM((CHUNK, D), _f32),
            pltpu.VMEM((CHUNK, D), _f32),
            pltpu.VMEM((CHUNK, D), _f32),
            pltpu.VMEM((CHUNK, D), _f32),
            pltpu.VMEM((CHUNK, 16), _f32),
            pltpu.VMEM((CHUNK, 16), _f32),
        ] + [pltpu.SemaphoreType.DMA] * 6,
    )
    def _sc_logits(ln, s1, r1, out, sidx_m, ridx_m, rs0, rs1, rr0, rr1,
                   ov0, ov1, *sems):
        rs = (rs0, rs1)
        rr = (rr0, rr1)
        ov = (ov0, ov1)
        gs = sems[0:2]
        gr = sems[2:4]
        oo = sems[4:6]
        c = lax.axis_index("c")
        tile = lax.axis_index("s")
        w = tile * 2 + c
        row0 = w * L_CHUNKS
        ebase0 = row0 * CHUNK

        pltpu.sync_copy(s1.at[pl.ds(row0, L_CHUNKS)], sidx_m)
        pltpu.sync_copy(r1.at[pl.ds(row0, L_CHUNKS)], ridx_m)
        pltpu.async_copy(ln.at[sidx_m.at[0]], rs[0], gs[0])
        pltpu.async_copy(ln.at[ridx_m.at[0]], rr[0], gr[0])

        def outer(it, _):
            for b in range(2):
                j = it * 2 + b
                pltpu.make_async_copy(ln.at[sidx_m.at[j]], rs[b], gs[b]).wait()
                pltpu.make_async_copy(ln.at[ridx_m.at[j]], rr[b], gr[b]).wait()

                @pl.when(j + 1 < L_CHUNKS)
                def _():
                    pltpu.async_copy(ln.at[sidx_m.at[j + 1]], rs[b ^ 1],
                                     gs[b ^ 1])
                    pltpu.async_copy(ln.at[ridx_m.at[j + 1]], rr[b ^ 1],
                                     gr[b ^ 1])

                @pl.when(j >= 2)
                def _():
                    pltpu.make_async_copy(
                        ov[b], out.at[pl.ds(ebase0, CHUNK)], oo[b]).wait()

                def row(i, _):
                    acc = rs[b][i, pl.ds(0, 16)] * rr[b][i, pl.ds(0, 16)]
                    for t in range(1, D // 16):
                        acc = acc + (rs[b][i, pl.ds(t * 16, 16)]
                                     * rr[b][i, pl.ds(t * 16, 16)])
                    ov[b][i, :] = acc
                    return 0
                lax.fori_loop(0, CHUNK, row, 0)
                pltpu.async_copy(
                    ov[b], out.at[pl.ds(ebase0 + j * CHUNK, CHUNK)], oo[b])
            return 0
        lax.fori_loop(0, L_CHUNKS // 2, outer, 0)
        for b in range(2):
            pltpu.make_async_copy(
                ov[b], out.at[pl.ds(ebase0, CHUNK)], oo[b]).wait()

    return _sc_prep, _sc_agg, _sc_logits


def _tc_layer0_body(x0_ref, ds1_ref, dr1_ref, ds2_ref, dr2_ref,
                    W1_ref, b1_ref, W2_ref, b2_ref,
                    h1_ref, h2_ref, is1_ref, is2_ref, ir1_ref, ir2_ref):
    x0 = x0_ref[...]
    shape = x0.shape
    is1 = jnp.broadcast_to(lax.rsqrt(ds1_ref[:, :1] + 1.0), shape)
    is2 = jnp.broadcast_to(lax.rsqrt(ds2_ref[:, :1] + 1.0), shape)
    ir1 = jnp.broadcast_to(lax.rsqrt(dr1_ref[:, :1] + 1.0), shape)
    ir2 = jnp.broadcast_to(lax.rsqrt(dr2_ref[:, :1] + 1.0), shape)
    h1_ref[...] = (jnp.dot(x0, W1_ref[...], precision=HI) + b1_ref[...]) * is1
    h2_ref[...] = (jnp.dot(x0, W2_ref[...], precision=HI) + b2_ref[...]) * is2
    is1_ref[...] = is1
    is2_ref[...] = is2
    ir1_ref[...] = ir1
    ir2_ref[...] = ir2


_BLK = 512
_GRID = N_PAD // _BLK


def _rows(shape=(_BLK, D)):
    return pl.BlockSpec(shape, lambda i: (i, 0))


def _full(shape):
    return pl.BlockSpec(shape, lambda i: (0, 0))


_tc_layer0 = pl.pallas_call(
    _tc_layer0_body,
    grid=(_GRID,),
    in_specs=[_rows(), _rows((_BLK, 16)), _rows((_BLK, 16)), _rows((_BLK, 16)),
              _rows((_BLK, 16)), _full((D, D)), _full((1, D)), _full((D, D)),
              _full((1, D))],
    out_specs=[_rows(), _rows(), _rows(), _rows(), _rows(), _rows()],
    out_shape=[jax.ShapeDtypeStruct((N_PAD, D), _f32)] * 6,
)


def _tc_layer_body(o1_ref, o2_ref, h1_ref, h2_ref, ir1_ref, ir2_ref,
                   is1_ref, is2_ref, WmA_ref, WmG_ref, bm_ref,
                   W1n_ref, b1n_ref, W2n_ref, b2n_ref,
                   h1n_ref, h2n_ref):
    a = (o1_ref[...] + h1_ref[...]) * ir1_ref[...]
    g = (o2_ref[...] + h2_ref[...]) * ir2_ref[...]
    xn = jnp.maximum(jnp.dot(a, WmA_ref[...], precision=HI)
                     + jnp.dot(g, WmG_ref[...], precision=HI) + bm_ref[...], 0.0)
    h1n_ref[...] = (jnp.dot(xn, W1n_ref[...], precision=HI) + b1n_ref[...]) * is1_ref[...]
    h2n_ref[...] = (jnp.dot(xn, W2n_ref[...], precision=HI) + b2n_ref[...]) * is2_ref[...]


_tc_layer = pl.pallas_call(
    _tc_layer_body,
    grid=(_GRID,),
    in_specs=[_rows()] * 8 + [_full((D, D)), _full((D, D)), _full((1, D)),
                              _full((D, D)), _full((1, D)), _full((D, D)),
                              _full((1, D))],
    out_specs=[_rows(), _rows()],
    out_shape=[jax.ShapeDtypeStruct((N_PAD, D), _f32)] * 2,
)


def _tc_final_body(o1_ref, o2_ref, h1_ref, h2_ref, ir1_ref, ir2_ref,
                   WmA_ref, WmG_ref, bm_ref, Wl_ref, bl_ref,
                   x_ref, ln_ref):
    a = (o1_ref[...] + h1_ref[...]) * ir1_ref[...]
    g = (o2_ref[...] + h2_ref[...]) * ir2_ref[...]
    xn = jnp.maximum(jnp.dot(a, WmA_ref[...], precision=HI)
                     + jnp.dot(g, WmG_ref[...], precision=HI) + bm_ref[...], 0.0)
    x_ref[...] = xn
    ln_ref[...] = jnp.dot(xn, Wl_ref[...], precision=HI) + bl_ref[...]


_tc_final = pl.pallas_call(
    _tc_final_body,
    grid=(_GRID,),
    in_specs=[_rows()] * 6 + [_full((D, D)), _full((D, D)), _full((1, D)),
                              _full((D, D)), _full((1, D))],
    out_specs=[_rows(), _rows()],
    out_shape=[jax.ShapeDtypeStruct((N_PAD, D), _f32)] * 2,
)


def _tc_eval_body(x_ref, We_ref, be_ref, Wo_ref, bo_ref, v_ref):
    x = x_ref[...]
    pid = lax.broadcasted_iota(jnp.int32, (D, N_PAD), 0)
    nid = lax.broadcasted_iota(jnp.int32, (D, N_PAD), 1)
    seg = (nid // (N // P) == pid) & (nid % (N // P) != 0)
    v = jnp.dot(seg.astype(_f32), x, precision=HI) * (1.0 / (N // P - 1))
    We = We_ref[...]
    be = be_ref[...]
    for i in range(N_EVAL):
        v = jnp.maximum(
            jnp.dot(v, We[i * D:(i + 1) * D, :], precision=HI) + be[i:i + 1, :],
            0.0)
    v_ref[...] = jnp.tanh(jnp.dot(v, Wo_ref[...], precision=HI) + bo_ref[...])


_tc_eval = pl.pallas_call(
    _tc_eval_body,
    out_shape=jax.ShapeDtypeStruct((D, D), _f32),
)


def _tc_lsum_body(pv_ref, out_ref):
    out_ref[...] = jnp.sum(pv_ref[...], axis=1, keepdims=True)


_tc_lsum = pl.pallas_call(
    _tc_lsum_body,
    grid=(128,),
    in_specs=[pl.BlockSpec((E_PAD // 128, 16), lambda i: (i, 0))],
    out_specs=pl.BlockSpec((E_PAD // 128, 1), lambda i: (i, 0)),
    out_shape=jax.ShapeDtypeStruct((E_PAD, 1), _f32),
)


def kernel(nodes, senders, receivers, grid_senders, grid_receivers, n_node,
           embed, W_conv1, b_conv1, W_conv2, b_conv2, W_mix, b_mix,
           W_logits, b_logits, W_eval, b_eval, W_out, b_out):
    sc_prep, sc_agg, sc_logits = _sc_kernels()

    pad_e = jnp.full((E_PAD - E,), N, jnp.int32)
    s1 = jnp.concatenate([senders, pad_e]).reshape(-1, CHUNK)
    r1 = jnp.concatenate([receivers, pad_e]).reshape(-1, CHUNK)
    s2 = jnp.concatenate([grid_senders, pad_e]).reshape(-1, CHUNK)
    r2 = jnp.concatenate([grid_receivers, pad_e]).reshape(-1, CHUNK)
    nodes_p = jnp.concatenate([nodes, jnp.zeros((N_PAD - N,), jnp.int32)])

    deg_s1, deg_r1, deg_s2, deg_r2, x0 = sc_prep(nodes_p, s1, r1, s2, r2, embed)

    h1, h2, is1, is2, ir1, ir2 = _tc_layer0(
        x0, deg_s1, deg_r1, deg_s2, deg_r2,
        W_conv1[0], b_conv1[0].reshape(1, D), W_conv2[0], b_conv2[0].reshape(1, D))

    for i in range(N_GNN - 1):
        o1, o2 = sc_agg(h1, h2, s1, r1, s2, r2)
        h1, h2 = _tc_layer(
            o1, o2, h1, h2, ir1, ir2, is1, is2,
            W_mix[i, :D, :], W_mix[i, D:, :], b_mix[i].reshape(1, D),
            W_conv1[i + 1], b_conv1[i + 1].reshape(1, D),
            W_conv2[i + 1], b_conv2[i + 1].reshape(1, D))

    o1, o2 = sc_agg(h1, h2, s1, r1, s2, r2)
    x, ln = _tc_final(
        o1, o2, h1, h2, ir1, ir2,
        W_mix[6, :D, :], W_mix[6, D:, :], b_mix[6].reshape(1, D),
        W_logits, b_logits.reshape(1, D))

    pv = sc_logits(ln, s1, r1)
    logits = _tc_lsum(pv)[:E, 0]

    v = _tc_eval(x, W_eval.reshape(N_EVAL * D, D), b_eval,
                 jnp.pad(W_out, ((0, 0), (0, D - 1))),
                 jnp.pad(b_out.reshape(1, 1), ((0, 0), (0, D - 1))))
    return logits, v[:P, :1]


# speedup vs baseline: 1.0785x; 1.0780x over previous
import functools

import jax
import jax.numpy as jnp
from jax import lax
from jax.experimental import pallas as pl
from jax.experimental.pallas import tpu as pltpu
import jax.experimental.pallas.tpu_sc as plsc

N = 10000
E = 320000
P = 100
D = 128
N_GNN = 7
N_EVAL = 5

N_PAD = 10240
E_PAD = 327680
CHUNK = 128
N_CHUNKS = E_PAD // (16 * CHUNK)
ROWS_PER_TILE = N_PAD // 16
L_CHUNKS = E_PAD // (32 * CHUNK)
X_ROWS = N_PAD // 32
XCH = 64
HI = lax.Precision.HIGHEST

_f32 = jnp.float32
_i32 = jnp.int32


@functools.cache
def _sc_kernels():
    mesh = plsc.VectorSubcoreMesh(core_axis_name="c", subcore_axis_name="s",
                                  num_cores=2, num_subcores=16)

    @functools.partial(
        pl.kernel,
        out_type=[
            jax.ShapeDtypeStruct((N_PAD, 16), _f32),
            jax.ShapeDtypeStruct((N_PAD, 16), _f32),
            jax.ShapeDtypeStruct((N_PAD, 16), _f32),
            jax.ShapeDtypeStruct((N_PAD, 16), _f32),
            jax.ShapeDtypeStruct((N_PAD, D), _f32),
        ],
        mesh=mesh,
        scratch_types=[
            pltpu.VMEM_SHARED((N_PAD, 16), _f32),
            pltpu.VMEM_SHARED((N_PAD, 16), _f32),
            pltpu.VMEM((CHUNK, 16), _f32),
            pltpu.VMEM((CHUNK, 16), _f32),
        ] + [pltpu.VMEM((CHUNK,), _i32)] * 8 + [
            pltpu.VMEM((XCH,), _i32),
            pltpu.VMEM((XCH, D), _f32),
            pltpu.SemaphoreType.DMA,
        ] + [pltpu.SemaphoreType.DMA] * 16,
    )
    def _sc_prep(nodes, s1, r1, s2, r2, embed,
                 deg_s1, deg_r1, deg_s2, deg_r2, x0,
                 acc_a, acc_b, ones_v, zeros_v,
                 ia0, ia1, ia2, ia3, ib0, ib1, ib2, ib3,
                 nidx_v, xrows_v, sem, *sems):
        ia = (ia0, ia1, ia2, ia3)
        ib = (ib0, ib1, ib2, ib3)
        sia = sems[0:4]
        sib = sems[4:8]
        ssa = sems[8:12]
        ssb = sems[12:16]
        c = lax.axis_index("c")
        tile = lax.axis_index("s")

        def fill(i, _):
            ones_v[i, :] = jnp.ones((16,), _f32)
            zeros_v[i, :] = jnp.zeros((16,), _f32)
            return 0
        lax.fori_loop(0, CHUNK, fill, 0)

        base_r = tile * ROWS_PER_TILE
        for k in range(ROWS_PER_TILE // CHUNK):
            pltpu.sync_copy(zeros_v, acc_a.at[pl.ds(base_r + k * CHUNK, CHUNK)])
            pltpu.sync_copy(zeros_v, acc_b.at[pl.ds(base_r + k * CHUNK, CHUNK)])
        plsc.subcore_barrier()

        def deg_pass(sref, rref):
            ebase0 = tile * (N_CHUNKS * CHUNK)

            def fire_idx(j, q):
                pltpu.async_copy(sref.at[pl.ds(ebase0 + j * CHUNK, CHUNK)],
                                 ia[q], sia[q])
                pltpu.async_copy(rref.at[pl.ds(ebase0 + j * CHUNK, CHUNK)],
                                 ib[q], sib[q])

            def wait_idx(j, q):
                pltpu.make_async_copy(
                    sref.at[pl.ds(ebase0 + j * CHUNK, CHUNK)], ia[q],
                    sia[q]).wait()
                pltpu.make_async_copy(
                    rref.at[pl.ds(ebase0 + j * CHUNK, CHUNK)], ib[q],
                    sib[q]).wait()

            fire_idx(0, 0)
            fire_idx(1, 1)

            def outer(it, _):
                for q4 in range(4):
                    j = it * 4 + q4
                    wait_idx(j, q4)
                    pltpu.async_copy(ones_v, acc_a.at[ia[q4]], ssa[q4],
                                     add=True)
                    pltpu.async_copy(ones_v, acc_b.at[ib[q4]], ssb[q4],
                                     add=True)

                    @pl.when(j >= 2)
                    def _():
                        pltpu.make_async_copy(
                            ones_v, acc_a.at[ia[q4 ^ 2]], ssa[q4 ^ 2]).wait()
                        pltpu.make_async_copy(
                            ones_v, acc_b.at[ib[q4 ^ 2]], ssb[q4 ^ 2]).wait()

                    @pl.when(j + 2 < N_CHUNKS)
                    def _():
                        fire_idx(j + 2, q4 ^ 2)
                return 0
            lax.fori_loop(0, N_CHUNKS // 4, outer, 0)
            for q4 in (2, 3):
                pltpu.make_async_copy(ones_v, acc_a.at[ia[q4]], ssa[q4]).wait()
                pltpu.make_async_copy(ones_v, acc_b.at[ib[q4]], ssb[q4]).wait()

        @pl.when(c == 0)
        def _():
            deg_pass(s1, r1)

        @pl.when(c == 1)
        def _():
            deg_pass(s2, r2)

        plsc.subcore_barrier()

        @pl.when(c == 0)
        def _():
            pltpu.sync_copy(acc_a.at[pl.ds(base_r, ROWS_PER_TILE)],
                            deg_s1.at[pl.ds(base_r, ROWS_PER_TILE)])
            pltpu.sync_copy(acc_b.at[pl.ds(base_r, ROWS_PER_TILE)],
                            deg_r1.at[pl.ds(base_r, ROWS_PER_TILE)])

        @pl.when(c == 1)
        def _():
            pltpu.sync_copy(acc_a.at[pl.ds(base_r, ROWS_PER_TILE)],
                            deg_s2.at[pl.ds(base_r, ROWS_PER_TILE)])
            pltpu.sync_copy(acc_b.at[pl.ds(base_r, ROWS_PER_TILE)],
                            deg_r2.at[pl.ds(base_r, ROWS_PER_TILE)])

        w = tile * 2 + c
        for j in range(X_ROWS // XCH):
            nbase = w * X_ROWS + j * XCH
            pltpu.sync_copy(nodes.at[pl.ds(nbase, XCH)], nidx_v)
            pltpu.async_copy(embed.at[nidx_v], xrows_v, sem).wait()
            pltpu.sync_copy(xrows_v, x0.at[pl.ds(nbase, XCH)])

    @functools.partial(
        pl.kernel,
        out_type=[
            jax.ShapeDtypeStruct((N_PAD, D), _f32),
            jax.ShapeDtypeStruct((N_PAD, D), _f32),
        ],
        mesh=mesh,
        scratch_types=[
            pltpu.VMEM_SHARED((N_PAD, D), _f32),
            pltpu.VMEM((CHUNK, D), _f32),
            pltpu.VMEM((CHUNK, D), _f32),
        ] + [pltpu.VMEM((CHUNK,), _i32)] * 4 + [
            pltpu.SemaphoreType.DMA,
        ] * 6,
    )
    def _sc_agg(h1, h2, s1, r1, s2, r2, o1, o2,
                acc, rw0, rw1, is0, is1_, ir0, ir1_, *sems):
        rows = (rw0, rw1)
        isx = (is0, is1_)
        irx = (ir0, ir1_)
        sis = sems[0:2]
        sir = sems[2:4]
        sg = sems[4:6]
        c = lax.axis_index("c")
        tile = lax.axis_index("s")

        def fill(i, _):
            for t in range(D // 16):
                rw0[i, pl.ds(t * 16, 16)] = jnp.zeros((16,), _f32)
            return 0
        lax.fori_loop(0, CHUNK, fill, 0)

        base_r = tile * ROWS_PER_TILE
        for k in range(ROWS_PER_TILE // CHUNK):
            pltpu.sync_copy(rw0, acc.at[pl.ds(base_r + k * CHUNK, CHUNK)])
        plsc.subcore_barrier()

        def agg(href, sref, rref):
            ebase0 = tile * (N_CHUNKS * CHUNK)

            def fire_idx(j, b):
                pltpu.async_copy(sref.at[pl.ds(ebase0 + j * CHUNK, CHUNK)],
                                 isx[b], sis[b])
                pltpu.async_copy(rref.at[pl.ds(ebase0 + j * CHUNK, CHUNK)],
                                 irx[b], sir[b])

            def wait_idx(j, b):
                pltpu.make_async_copy(
                    sref.at[pl.ds(ebase0 + j * CHUNK, CHUNK)], isx[b],
                    sis[b]).wait()
                pltpu.make_async_copy(
                    rref.at[pl.ds(ebase0 + j * CHUNK, CHUNK)], irx[b],
                    sir[b]).wait()

            fire_idx(0, 0)
            fire_idx(1, 1)
            wait_idx(0, 0)
            pltpu.async_copy(href.at[isx[0]], rows[0], sg[0])

            def outer(it, _):
                for b in range(2):
                    j = it * 2 + b

                    @pl.when(j + 1 < N_CHUNKS)
                    def _():
                        wait_idx(j + 1, b ^ 1)
                        pltpu.async_copy(href.at[isx[b ^ 1]], rows[b ^ 1],
                                         sg[b ^ 1])
                    pltpu.make_async_copy(href.at[isx[b]], rows[b],
                                          sg[b]).wait()
                    pltpu.sync_copy(rows[b], acc.at[irx[b]], add=True)

                    @pl.when(j + 2 < N_CHUNKS)
                    def _():
                        fire_idx(j + 2, b)
                return 0
            lax.fori_loop(0, N_CHUNKS // 2, outer, 0)

        @pl.when(c == 0)
        def _():
            agg(h1, s1, r1)

        @pl.when(c == 1)
        def _():
            agg(h2, s2, r2)

        plsc.subcore_barrier()

        @pl.when(c == 0)
        def _():
            pltpu.sync_copy(acc.at[pl.ds(base_r, ROWS_PER_TILE)],
                            o1.at[pl.ds(base_r, ROWS_PER_TILE)])

        @pl.when(c == 1)
        def _():
            pltpu.sync_copy(acc.at[pl.ds(base_r, ROWS_PER_TILE)],
                            o2.at[pl.ds(base_r, ROWS_PER_TILE)])

    @functools.partial(
        pl.kernel,
        out_type=jax.ShapeDtypeStruct((E_PAD, 16), _f32),
        mesh=mesh,
        scratch_types=[
            pltpu.VMEM((CHUNK, D), _f32),
            pltpu.VMEM((CHUNK, D), _f32),
            pltpu.VMEM((CHUNK, D), _f32),
            pltpu.VMEM((CHUNK, D), _f32),
            pltpu.VMEM((CHUNK, 16), _f32),
            pltpu.VMEM((CHUNK, 16), _f32),
        ] + [pltpu.VMEM((CHUNK,), _i32)] * 4 + [
            pltpu.SemaphoreType.DMA,
        ] * 10,
    )
    def _sc_logits(ln, s1, r1, out, rs0, rs1, rr0, rr1, ov0, ov1,
                   is0, is1_, ir0, ir1_, *sems):
        rs = (rs0, rs1)
        rr = (rr0, rr1)
        ov = (ov0, ov1)
        isx = (is0, is1_)
        irx = (ir0, ir1_)
        sis = sems[0:2]
        sir = sems[2:4]
        gs = sems[4:6]
        gr = sems[6:8]
        oo = sems[8:10]
        c = lax.axis_index("c")
        tile = lax.axis_index("s")
        w = tile * 2 + c
        ebase0 = w * (L_CHUNKS * CHUNK)

        def fire_idx(j, b):
            pltpu.async_copy(s1.at[pl.ds(ebase0 + j * CHUNK, CHUNK)],
                             isx[b], sis[b])
            pltpu.async_copy(r1.at[pl.ds(ebase0 + j * CHUNK, CHUNK)],
                             irx[b], sir[b])

        def wait_idx(j, b):
            pltpu.make_async_copy(
                s1.at[pl.ds(ebase0 + j * CHUNK, CHUNK)], isx[b], sis[b]).wait()
            pltpu.make_async_copy(
                r1.at[pl.ds(ebase0 + j * CHUNK, CHUNK)], irx[b], sir[b]).wait()

        def fire_gathers(b):
            pltpu.async_copy(ln.at[isx[b]], rs[b], gs[b])
            pltpu.async_copy(ln.at[irx[b]], rr[b], gr[b])

        def wait_gathers(b):
            pltpu.make_async_copy(ln.at[isx[b]], rs[b], gs[b]).wait()
            pltpu.make_async_copy(ln.at[irx[b]], rr[b], gr[b]).wait()

        fire_idx(0, 0)
        fire_idx(1, 1)
        wait_idx(0, 0)
        fire_gathers(0)

        def outer(it, _):
            for b in range(2):
                j = it * 2 + b

                @pl.when(j + 1 < L_CHUNKS)
                def _():
                    wait_idx(j + 1, b ^ 1)
                    fire_gathers(b ^ 1)
                wait_gathers(b)

                @pl.when(j + 2 < L_CHUNKS)
                def _():
                    fire_idx(j + 2, b)

                @pl.when(j >= 2)
                def _():
                    pltpu.make_async_copy(
                        ov[b], out.at[pl.ds(ebase0, CHUNK)], oo[b]).wait()

                def row(i, _):
                    acc = rs[b][i, pl.ds(0, 16)] * rr[b][i, pl.ds(0, 16)]
                    for t in range(1, D // 16):
                        acc = acc + (rs[b][i, pl.ds(t * 16, 16)]
                                     * rr[b][i, pl.ds(t * 16, 16)])
                    ov[b][i, :] = acc
                    return 0
                lax.fori_loop(0, CHUNK, row, 0)
                pltpu.async_copy(
                    ov[b], out.at[pl.ds(ebase0 + j * CHUNK, CHUNK)], oo[b])
            return 0
        lax.fori_loop(0, L_CHUNKS // 2, outer, 0)
        for b in range(2):
            pltpu.make_async_copy(
                ov[b], out.at[pl.ds(ebase0, CHUNK)], oo[b]).wait()

    return _sc_prep, _sc_agg, _sc_logits


def _tc_layer0_body(x0_ref, ds1_ref, dr1_ref, ds2_ref, dr2_ref,
                    W1_ref, b1_ref, W2_ref, b2_ref,
                    h1_ref, h2_ref, is1_ref, is2_ref, ir1_ref, ir2_ref):
    x0 = x0_ref[...]
    shape = x0.shape
    is1 = jnp.broadcast_to(lax.rsqrt(ds1_ref[:, :1] + 1.0), shape)
    is2 = jnp.broadcast_to(lax.rsqrt(ds2_ref[:, :1] + 1.0), shape)
    ir1 = jnp.broadcast_to(lax.rsqrt(dr1_ref[:, :1] + 1.0), shape)
    ir2 = jnp.broadcast_to(lax.rsqrt(dr2_ref[:, :1] + 1.0), shape)
    h1_ref[...] = (jnp.dot(x0, W1_ref[...], precision=HI) + b1_ref[...]) * is1
    h2_ref[...] = (jnp.dot(x0, W2_ref[...], precision=HI) + b2_ref[...]) * is2
    is1_ref[...] = is1
    is2_ref[...] = is2
    ir1_ref[...] = ir1
    ir2_ref[...] = ir2


_BLK = 512
_GRID = N_PAD // _BLK


def _rows(shape=(_BLK, D)):
    return pl.BlockSpec(shape, lambda i: (i, 0))


def _full(shape):
    return pl.BlockSpec(shape, lambda i: (0, 0))


_tc_layer0 = pl.pallas_call(
    _tc_layer0_body,
    grid=(_GRID,),
    in_specs=[_rows(), _rows((_BLK, 16)), _rows((_BLK, 16)), _rows((_BLK, 16)),
              _rows((_BLK, 16)), _full((D, D)), _full((1, D)), _full((D, D)),
              _full((1, D))],
    out_specs=[_rows(), _rows(), _rows(), _rows(), _rows(), _rows()],
    out_shape=[jax.ShapeDtypeStruct((N_PAD, D), _f32)] * 6,
)


def _tc_layer_body(o1_ref, o2_ref, h1_ref, h2_ref, ir1_ref, ir2_ref,
                   is1_ref, is2_ref, WmA_ref, WmG_ref, bm_ref,
                   W1n_ref, b1n_ref, W2n_ref, b2n_ref,
                   h1n_ref, h2n_ref):
    a = (o1_ref[...] + h1_ref[...]) * ir1_ref[...]
    g = (o2_ref[...] + h2_ref[...]) * ir2_ref[...]
    xn = jnp.maximum(jnp.dot(a, WmA_ref[...], precision=HI)
                     + jnp.dot(g, WmG_ref[...], precision=HI) + bm_ref[...], 0.0)
    h1n_ref[...] = (jnp.dot(xn, W1n_ref[...], precision=HI) + b1n_ref[...]) * is1_ref[...]
    h2n_ref[...] = (jnp.dot(xn, W2n_ref[...], precision=HI) + b2n_ref[...]) * is2_ref[...]


_tc_layer = pl.pallas_call(
    _tc_layer_body,
    grid=(_GRID,),
    in_specs=[_rows()] * 8 + [_full((D, D)), _full((D, D)), _full((1, D)),
                              _full((D, D)), _full((1, D)), _full((D, D)),
                              _full((1, D))],
    out_specs=[_rows(), _rows()],
    out_shape=[jax.ShapeDtypeStruct((N_PAD, D), _f32)] * 2,
)


def _tc_final_body(o1_ref, o2_ref, h1_ref, h2_ref, ir1_ref, ir2_ref,
                   WmA_ref, WmG_ref, bm_ref, Wl_ref, bl_ref,
                   x_ref, ln_ref):
    a = (o1_ref[...] + h1_ref[...]) * ir1_ref[...]
    g = (o2_ref[...] + h2_ref[...]) * ir2_ref[...]
    xn = jnp.maximum(jnp.dot(a, WmA_ref[...], precision=HI)
                     + jnp.dot(g, WmG_ref[...], precision=HI) + bm_ref[...], 0.0)
    x_ref[...] = xn
    ln_ref[...] = jnp.dot(xn, Wl_ref[...], precision=HI) + bl_ref[...]


_tc_final = pl.pallas_call(
    _tc_final_body,
    grid=(_GRID,),
    in_specs=[_rows()] * 6 + [_full((D, D)), _full((D, D)), _full((1, D)),
                              _full((D, D)), _full((1, D))],
    out_specs=[_rows(), _rows()],
    out_shape=[jax.ShapeDtypeStruct((N_PAD, D), _f32)] * 2,
)


def _tc_eval_body(x_ref, We_ref, be_ref, Wo_ref, bo_ref, v_ref):
    x = x_ref[...]
    pid = lax.broadcasted_iota(jnp.int32, (D, N_PAD), 0)
    nid = lax.broadcasted_iota(jnp.int32, (D, N_PAD), 1)
    seg = (nid // (N // P) == pid) & (nid % (N // P) != 0)
    v = jnp.dot(seg.astype(_f32), x, precision=HI) * (1.0 / (N // P - 1))
    We = We_ref[...]
    be = be_ref[...]
    for i in range(N_EVAL):
        v = jnp.maximum(
            jnp.dot(v, We[i * D:(i + 1) * D, :], precision=HI) + be[i:i + 1, :],
            0.0)
    v_ref[...] = jnp.tanh(jnp.dot(v, Wo_ref[...], precision=HI) + bo_ref[...])


_tc_eval = pl.pallas_call(
    _tc_eval_body,
    out_shape=jax.ShapeDtypeStruct((D, D), _f32),
)


def _tc_lsum_body(pv_ref, out_ref):
    out_ref[...] = jnp.sum(pv_ref[...], axis=1, keepdims=True)


_tc_lsum = pl.pallas_call(
    _tc_lsum_body,
    grid=(128,),
    in_specs=[pl.BlockSpec((E_PAD // 128, 16), lambda i: (i, 0))],
    out_specs=pl.BlockSpec((E_PAD // 128, 1), lambda i: (i, 0)),
    out_shape=jax.ShapeDtypeStruct((E_PAD, 1), _f32),
)


def kernel(nodes, senders, receivers, grid_senders, grid_receivers, n_node,
           embed, W_conv1, b_conv1, W_conv2, b_conv2, W_mix, b_mix,
           W_logits, b_logits, W_eval, b_eval, W_out, b_out):
    sc_prep, sc_agg, sc_logits = _sc_kernels()

    pad_e = jnp.full((E_PAD - E,), N, jnp.int32)
    s1 = jnp.concatenate([senders, pad_e])
    r1 = jnp.concatenate([receivers, pad_e])
    s2 = jnp.concatenate([grid_senders, pad_e])
    r2 = jnp.concatenate([grid_receivers, pad_e])
    nodes_p = jnp.concatenate([nodes, jnp.zeros((N_PAD - N,), jnp.int32)])

    deg_s1, deg_r1, deg_s2, deg_r2, x0 = sc_prep(nodes_p, s1, r1, s2, r2, embed)

    h1, h2, is1, is2, ir1, ir2 = _tc_layer0(
        x0, deg_s1, deg_r1, deg_s2, deg_r2,
        W_conv1[0], b_conv1[0].reshape(1, D), W_conv2[0], b_conv2[0].reshape(1, D))

    for i in range(N_GNN - 1):
        o1, o2 = sc_agg(h1, h2, s1, r1, s2, r2)
        h1, h2 = _tc_layer(
            o1, o2, h1, h2, ir1, ir2, is1, is2,
            W_mix[i, :D, :], W_mix[i, D:, :], b_mix[i].reshape(1, D),
            W_conv1[i + 1], b_conv1[i + 1].reshape(1, D),
            W_conv2[i + 1], b_conv2[i + 1].reshape(1, D))

    o1, o2 = sc_agg(h1, h2, s1, r1, s2, r2)
    x, ln = _tc_final(
        o1, o2, h1, h2, ir1, ir2,
        W_mix[6, :D, :], W_mix[6, D:, :], b_mix[6].reshape(1, D),
        W_logits, b_logits.reshape(1, D))

    pv = sc_logits(ln, s1, r1)
    logits = _tc_lsum(pv)[:E, 0]

    v = _tc_eval(x, W_eval.reshape(N_EVAL * D, D), b_eval,
                 jnp.pad(W_out, ((0, 0), (0, D - 1))),
                 jnp.pad(b_out.reshape(1, 1), ((0, 0), (0, D - 1))))
    return logits, v[:P, :1]
